# baseline (speedup 1.0000x reference)
import functools

import jax
import jax.numpy as jnp
from jax import lax
from jax.experimental import pallas as pl
from jax.experimental.pallas import tpu as pltpu
from jax.experimental.pallas import tpu_sc as plsc

_N = 10000
_E = 320000
_NSC = 2
_NTILE = 16
_NW = _NSC * _NTILE
_RPT = 624
_TAIL = _N - _NTILE * _RPT
_C = 96
_NCH = 106
_NCHP = _NCH + 2
_EWP = _NCHP * _C
_EP = _NW * _EWP


def _spmm_body(D, m_hbm, col_hbm, row_hbm, w_hbm, out_hbm,
               colb, wb, rowb, rows0, rows1, acc,
               semg0, semg1, semr0, semr1):
    c = lax.axis_index("c")
    s = lax.axis_index("s")
    wid = c * _NTILE + s

    pltpu.sync_copy(col_hbm.at[wid], colb)
    pltpu.sync_copy(w_hbm.at[wid], wb)

    def zrow(i, carry):
        for j in range(D // 16):
            rows0[i, pl.ds(j * 16, 16)] = jnp.zeros((16,), jnp.float32)
        return carry

    lax.fori_loop(0, _C, zrow, 0)
    base = s * _RPT
    for k in range(_RPT // _C):
        pltpu.sync_copy(rows0, acc.at[pl.ds(base + k * _C, _C)])
    pltpu.sync_copy(rows0.at[pl.ds(0, _RPT - (_RPT // _C) * _C)],
                    acc.at[pl.ds(base + (_RPT // _C) * _C,
                                 _RPT - (_RPT // _C) * _C)])

    @pl.when(s == _NTILE - 1)
    def _zero_tail():
        pltpu.sync_copy(rows0.at[pl.ds(0, _TAIL)],
                        acc.at[pl.ds(_NTILE * _RPT, _TAIL)])

    pltpu.async_copy(row_hbm.at[wid, 0], rowb.at[0], semr0)
    pltpu.async_copy(row_hbm.at[wid, 1], rowb.at[1], semr1)
    pltpu.async_copy(m_hbm.at[colb.at[pl.ds(0, _C)]], rows0, semg0)
    pltpu.async_copy(m_hbm.at[colb.at[pl.ds(_C, _C)]], rows1, semg1)
    plsc.subcore_barrier()

    def scale(buf, k):
        def body(g, inner):
            w16 = wb[pl.ds(k * _C + g * 16, 16)]
            for l in range(16):
                e = g * 16 + l
                wspl = jnp.full((16,), w16[l], jnp.float32)
                for j in range(D // 16):
                    buf[e, pl.ds(j * 16, 16)] = (
                        buf[e, pl.ds(j * 16, 16)] * wspl)
            return inner

        lax.fori_loop(0, _C // 16, body, 0)

    def half(k, rows, semg, slot, semr):
        pltpu.make_async_copy(
            m_hbm.at[colb.at[pl.ds(0, _C)]], rows, semg).wait()
        scale(rows, k)
        pltpu.make_async_copy(
            row_hbm.at[wid, 0], rowb.at[slot], semr).wait()
        pltpu.sync_copy(rows, acc.at[rowb.at[slot]], add=True)
        pltpu.async_copy(row_hbm.at[wid, k + 2], rowb.at[slot], semr)
        pltpu.async_copy(
            m_hbm.at[colb.at[pl.ds((k + 2) * _C, _C)]], rows, semg)

    def pair(j, carry):
        half(2 * j, rows0, semg0, 0, semr0)
        half(2 * j + 1, rows1, semg1, 1, semr1)
        return carry

    lax.fori_loop(0, _NCH // 2, pair, 0)
    pltpu.make_async_copy(m_hbm.at[colb.at[pl.ds(0, _C)]], rows0,
                          semg0).wait()
    pltpu.make_async_copy(m_hbm.at[colb.at[pl.ds(0, _C)]], rows1,
                          semg1).wait()
    pltpu.make_async_copy(row_hbm.at[wid, 0], rowb.at[0], semr0).wait()
    pltpu.make_async_copy(row_hbm.at[wid, 1], rowb.at[1], semr1).wait()
    plsc.subcore_barrier()

    pltpu.sync_copy(acc.at[pl.ds(s * _RPT, _RPT)],
                    out_hbm.at[c, pl.ds(s * _RPT, _RPT)])

    @pl.when(s == _NTILE - 1)
    def _write_tail():
        pltpu.sync_copy(acc.at[pl.ds(_NTILE * _RPT, _TAIL)],
                        out_hbm.at[c, pl.ds(_NTILE * _RPT, _TAIL)])


@functools.cache
def _make_spmm(D):
    mesh = plsc.VectorSubcoreMesh(core_axis_name="c", subcore_axis_name="s")
    return pl.kernel(
        functools.partial(_spmm_body, D),
        out_type=jax.ShapeDtypeStruct((_NSC, _N, D), jnp.float32),
        mesh=mesh,
        scratch_types=[
            pltpu.VMEM((_EWP,), jnp.int32),
            pltpu.VMEM((_EWP,), jnp.float32),
            pltpu.VMEM((2, _C), jnp.int32),
            pltpu.VMEM((_C, D), jnp.float32),
            pltpu.VMEM((_C, D), jnp.float32),
            pltpu.VMEM_SHARED((_N, D), jnp.float32),
            pltpu.SemaphoreType.DMA,
            pltpu.SemaphoreType.DMA,
            pltpu.SemaphoreType.DMA,
            pltpu.SemaphoreType.DMA,
        ],
        name=f"gcn_spmm_d{D}",
    )


def _matmul_body(x_ref, w_ref, o_ref):
    o_ref[...] = jnp.dot(x_ref[...], w_ref[...],
                         preferred_element_type=jnp.float32)


def _fused_body(p0_ref, p1_ref, b_ref, w_ref, o_ref):
    h = jnp.maximum(p0_ref[...] + p1_ref[...] + b_ref[...], 0.0)
    o_ref[...] = jnp.dot(h, w_ref[...], preferred_element_type=jnp.float32)


def _combine_relu_body(p0_ref, p1_ref, b_ref, o_ref):
    o_ref[...] = jnp.maximum(p0_ref[...] + p1_ref[...] + b_ref[...], 0.0)


def _final_body(p0_ref, p1_ref, w_ref, b_ref, o_ref):
    o_ref[...] = jnp.dot(p0_ref[...] + p1_ref[...], w_ref[...],
                         preferred_element_type=jnp.float32) + b_ref[...]


_BLK = 1000


def _matmul(x, W):
    K, M = W.shape
    return pl.pallas_call(
        _matmul_body,
        grid=(_N // _BLK,),
        in_specs=[
            pl.BlockSpec((_BLK, K), lambda i: (i, 0)),
            pl.BlockSpec((K, M), lambda i: (0, 0)),
        ],
        out_specs=pl.BlockSpec((_BLK, M), lambda i: (i, 0)),
        out_shape=jax.ShapeDtypeStruct((_N, M), jnp.float32),
    )(x, W)


def _fused(p0, p1, b, W):
    K, M = W.shape
    return pl.pallas_call(
        _fused_body,
        grid=(_N // _BLK,),
        in_specs=[
            pl.BlockSpec((_BLK, K), lambda i: (i, 0)),
            pl.BlockSpec((_BLK, K), lambda i: (i, 0)),
            pl.BlockSpec((1, K), lambda i: (0, 0)),
            pl.BlockSpec((K, M), lambda i: (0, 0)),
        ],
        out_specs=pl.BlockSpec((_BLK, M), lambda i: (i, 0)),
        out_shape=jax.ShapeDtypeStruct((_N, M), jnp.float32),
    )(p0, p1, b.reshape(1, K), W)


def _combine_relu(p0, p1, b):
    M = p0.shape[1]
    return pl.pallas_call(
        _combine_relu_body,
        grid=(_N // _BLK,),
        in_specs=[
            pl.BlockSpec((_BLK, M), lambda i: (i, 0)),
            pl.BlockSpec((_BLK, M), lambda i: (i, 0)),
            pl.BlockSpec((1, M), lambda i: (0, 0)),
        ],
        out_specs=pl.BlockSpec((_BLK, M), lambda i: (i, 0)),
        out_shape=jax.ShapeDtypeStruct((_N, M), jnp.float32),
    )(p0, p1, b.reshape(1, M))


def _final(p0, p1, W, b):
    K, M = W.shape
    return pl.pallas_call(
        _final_body,
        grid=(_N // _BLK,),
        in_specs=[
            pl.BlockSpec((_BLK, K), lambda i: (i, 0)),
            pl.BlockSpec((_BLK, K), lambda i: (i, 0)),
            pl.BlockSpec((K, M), lambda i: (0, 0)),
            pl.BlockSpec((1, M), lambda i: (0, 0)),
        ],
        out_specs=pl.BlockSpec((_BLK, M), lambda i: (i, 0)),
        out_shape=jax.ShapeDtypeStruct((_N, M), jnp.float32),
    )(p0, p1, W, b.reshape(1, M))


def kernel(x, edge_index, edge_weight, W0, b0, W1, b1, W2, b2):
    ew_proc = _NCH * _C
    pad = _NW * ew_proc - _E

    def _prep(a):
        a2 = jnp.pad(a, (0, pad)).reshape(_NW, ew_proc)
        return jnp.pad(a2, ((0, 0), (0, _EWP - ew_proc)))

    row = _prep(edge_index[0].astype(jnp.int32)).reshape(_NW, _NCHP, _C)
    col = _prep(edge_index[1].astype(jnp.int32))
    w = _prep(edge_weight.astype(jnp.float32))

    spmm128 = _make_spmm(128)

    t0 = _matmul(x, W0)
    p0 = spmm128(t0, col, row, w)
    t1 = _fused(p0[0], p0[1], b0, W1)
    p1 = spmm128(t1, col, row, w)
    t2 = _combine_relu(p1[0], p1[1], b1)
    p2 = spmm128(t2, col, row, w)
    return _final(p2[0], p2[1], W2, b2)
